# baseline (speedup 1.0000x reference)
import functools

import jax
import jax.numpy as jnp
from jax import lax
from jax.experimental import pallas as pl
from jax.experimental.pallas import tpu as pltpu
from jax.experimental.pallas import tpu_sc as plsc

_NC, _NS, _L = 2, 16, 16
_NW = _NC * _NS
_TAB_PAD = 352


@functools.lru_cache(maxsize=None)
def _build(nbatch, seqlen, R):
    B = nbatch * seqlen
    assert nbatch % _NW == 0
    rows_w = nbatch // _NW
    assert rows_w % R == 0
    n_chunks = rows_w // R
    C = R * seqlen
    assert C % _L == 0
    mesh = plsc.VectorSubcoreMesh(core_axis_name="c", subcore_axis_name="s")

    @functools.partial(
        pl.kernel,
        out_type=jax.ShapeDtypeStruct((nbatch, seqlen, 1, 4), jnp.float32),
        mesh=mesh,
        scratch_types=[
            pltpu.VMEM((_TAB_PAD,), jnp.float32),
            pltpu.VMEM((C,), jnp.int32),
            pltpu.VMEM((R, seqlen, 1, 4), jnp.float32),
        ],
        compiler_params=pltpu.CompilerParams(
            needs_layout_passes=False, use_tc_tiling_on_sc=False),
    )
    def k(ids_hbm, tab_hbm, out_hbm, tab_v, ids_v, out_v):
        wid = lax.axis_index("s") * _NC + lax.axis_index("c")
        row0 = wid * rows_w
        pltpu.sync_copy(tab_hbm, tab_v)
        iota = lax.broadcasted_iota(jnp.int32, (_L,), 0)
        zeros = jnp.zeros((_L,), jnp.int32)
        col_splats = [jnp.full((_L,), c, jnp.int32) for c in range(4)]

        def chunk_body(j, carry):
            rb = row0 + j * R
            pltpu.sync_copy(ids_hbm.at[pl.ds(rb * seqlen, C)], ids_v)

            @plsc.parallel_loop(0, C // _L, unroll=2)
            def it_body(t):
                ids16 = ids_v[pl.ds(t * _L, _L)]
                ids4 = ids16 * 4
                pos = t * _L + iota
                for c in range(4):
                    vals = plsc.load_gather(tab_v, [ids4 + c])
                    plsc.store_scatter(
                        out_v, [zeros, pos, zeros, col_splats[c]], vals)
            pltpu.sync_copy(out_v, out_hbm.at[pl.ds(rb, R)])
            return carry

        lax.fori_loop(0, n_chunks, chunk_body, 0)

    return k


def kernel(dna, embedding_table):
    nbatch, seqlen = dna.shape
    tab = jnp.pad(embedding_table.reshape(-1),
                  (0, _TAB_PAD - embedding_table.size))
    return _build(nbatch, seqlen, 64)(dna.reshape(-1), tab)


# speedup vs baseline: 46.3733x; 46.3733x over previous
import functools

import jax
import jax.numpy as jnp
from jax import lax
from jax.experimental import pallas as pl
from jax.experimental.pallas import tpu as pltpu
from jax.experimental.pallas import tpu_sc as plsc

_NC, _NS, _L = 2, 16, 16
_NW = _NC * _NS
_TAB_PAD = 352
_BLK = 128
_NBLK = 32


@functools.lru_cache(maxsize=None)
def _build(nbatch, seqlen):
    B = nbatch * seqlen
    nblk_s = nbatch // _BLK
    chunks_s = nblk_s // _NBLK
    n_chunks = seqlen * chunks_s
    per_w = n_chunks // _NW
    assert per_w * _NW == n_chunks and chunks_s * _NBLK == nblk_s
    C = _NBLK * _BLK
    mesh = plsc.VectorSubcoreMesh(core_axis_name="c", subcore_axis_name="s")

    @functools.partial(
        pl.kernel,
        out_type=jax.ShapeDtypeStruct((seqlen, nblk_s, 4, _BLK), jnp.float32),
        mesh=mesh,
        scratch_types=[
            pltpu.VMEM((_TAB_PAD,), jnp.float32),
            pltpu.VMEM((C,), jnp.int32),
            pltpu.VMEM((_NBLK, 4, _BLK), jnp.float32),
        ],
        compiler_params=pltpu.CompilerParams(
            needs_layout_passes=False, use_tc_tiling_on_sc=False),
    )
    def k(ids_hbm, tab_hbm, out_hbm, tab_v, ids_v, out_v):
        wid = lax.axis_index("s") * _NC + lax.axis_index("c")
        k0 = wid * per_w
        pltpu.sync_copy(tab_hbm, tab_v)

        def chunk_body(i, carry):
            kk = k0 + i
            s = kk // chunks_s
            cidx = kk % chunks_s
            blk0 = cidx * _NBLK
            p0 = s * nbatch + blk0 * _BLK
            pltpu.sync_copy(ids_hbm.at[pl.ds(p0, C)], ids_v)

            @plsc.parallel_loop(0, _NBLK, unroll=1)
            def blk_body(blk):
                for g in range(_BLK // _L):
                    ids16 = ids_v[pl.ds(blk * _BLK + g * _L, _L)]
                    ids4 = ids16 * 4
                    for c in range(4):
                        vals = plsc.load_gather(tab_v, [ids4 + c])
                        out_v[blk, c, pl.ds(g * _L, _L)] = vals
            pltpu.sync_copy(out_v, out_hbm.at[s, pl.ds(blk0, _NBLK)])
            return carry

        lax.fori_loop(0, per_w, chunk_body, 0)

    return k


def kernel(dna, embedding_table):
    nbatch, seqlen = dna.shape
    tab = jnp.pad(embedding_table.reshape(-1),
                  (0, _TAB_PAD - embedding_table.size))
    ids_t = dna.T.reshape(-1)
    y = _build(nbatch, seqlen)(ids_t, tab)
    return y.transpose(1, 3, 0, 2).reshape(nbatch, seqlen, 1, 4)
